# baseline (speedup 1.0000x reference)
import jax
import jax.numpy as jnp
from jax.experimental import pallas as pl
from jax.experimental.pallas import tpu as pltpu

NUM_RAAGS = 1000
EMBED_DIM = 128
SEQ_LEN = 512
BATCH = 1024
B_BLK = 16
HALF = BATCH // 2


def _tile_kernel(idx_ref, table_ref, out_ref, *, base):
    i = pl.program_id(0)
    for j in range(B_BLK):
        idx = idx_ref[base + i * B_BLK + j]
        row = table_ref[pl.ds(idx, 1), :]
        out_ref[j, :, :] = jnp.broadcast_to(row, (SEQ_LEN, EMBED_DIM))


def _half(idx, table, base):
    import functools
    grid_spec = pltpu.PrefetchScalarGridSpec(
        num_scalar_prefetch=1,
        grid=(HALF // B_BLK,),
        in_specs=[
            pl.BlockSpec((NUM_RAAGS, EMBED_DIM), lambda i, idx_ref: (0, 0)),
        ],
        out_specs=pl.BlockSpec(
            (B_BLK, SEQ_LEN, EMBED_DIM), lambda i, idx_ref: (i, 0, 0)
        ),
    )
    return pl.pallas_call(
        functools.partial(_tile_kernel, base=base),
        grid_spec=grid_spec,
        out_shape=jax.ShapeDtypeStruct((HALF, SEQ_LEN, EMBED_DIM), jnp.float32),
    )(idx, table)


def kernel(raag_embeddings, table):
    idx = raag_embeddings.reshape(BATCH)
    out0 = _half(idx, table, 0)
    out1 = _half(idx, table, HALF)
    return jnp.concatenate([out0, out1], axis=0)


# speedup vs baseline: 2.5159x; 2.5159x over previous
import functools

import jax
import jax.numpy as jnp
from jax import lax
from jax.experimental import pallas as pl
from jax.experimental.pallas import tpu as pltpu
from jax.experimental.pallas import tpu_sc as plsc

NUM_RAAGS = 1000
EMBED_DIM = 128
SEQ_LEN = 512
BATCH = 1024
B_BLK = 16

_INFO = plsc.get_sparse_core_info()
_NC = _INFO.num_cores
_NS = _INFO.num_subcores
_NW = _NC * _NS
_B_PER_W = BATCH // _NW


def _sc_gather(table_hbm, idx_hbm, out_hbm, idx_v, rows_v, sem):
    wid = lax.axis_index("s") * _NC + lax.axis_index("c")
    base = wid * _B_PER_W
    pltpu.sync_copy(idx_hbm.at[pl.ds(base, _B_PER_W)], idx_v)
    pltpu.async_copy(table_hbm.at[idx_v], rows_v, sem).wait()
    pltpu.sync_copy(rows_v, out_hbm.at[pl.ds(base, _B_PER_W)])


def _bcast_kernel(rows_ref, out_ref):
    out_ref[...] = jnp.broadcast_to(rows_ref[...][:, None, :], out_ref.shape)


def kernel(raag_embeddings, table):
    idx = raag_embeddings.reshape(BATCH)

    mesh = plsc.VectorSubcoreMesh(core_axis_name="c", subcore_axis_name="s", num_cores=1)
    gathered = pl.kernel(
        _sc_gather,
        mesh=mesh,
        out_type=jax.ShapeDtypeStruct((BATCH, EMBED_DIM), jnp.float32),
        scratch_types=[
            pltpu.VMEM((_B_PER_W,), jnp.int32),
            pltpu.VMEM((_B_PER_W, EMBED_DIM), jnp.float32),
            pltpu.SemaphoreType.DMA,
        ],
    )(table, idx)

    out = pl.pallas_call(
        _bcast_kernel,
        grid=(BATCH // B_BLK,),
        in_specs=[pl.BlockSpec((B_BLK, EMBED_DIM), lambda i: (i, 0))],
        out_specs=pl.BlockSpec((B_BLK, SEQ_LEN, EMBED_DIM), lambda i: (i, 0, 0)),
        out_shape=jax.ShapeDtypeStruct((BATCH, SEQ_LEN, EMBED_DIM), jnp.float32),
    )(gathered)
    return out


# speedup vs baseline: 2.5272x; 1.0045x over previous
import functools

import jax
import jax.numpy as jnp
from jax import lax
from jax.experimental import pallas as pl
from jax.experimental.pallas import tpu as pltpu
from jax.experimental.pallas import tpu_sc as plsc

NUM_RAAGS = 1000
EMBED_DIM = 128
SEQ_LEN = 512
BATCH = 1024
B_BLK = 16
SC_ROWS = 512
TC_ROWS = BATCH - SC_ROWS

_INFO = plsc.get_sparse_core_info()
_NC = _INFO.num_cores
_NS = _INFO.num_subcores
_NW = _NC * _NS
_B_PER_W = SC_ROWS // _NW


def _sc_gather(table_hbm, idx_hbm, out_hbm, idx_v, rows_v, sem):
    wid = lax.axis_index("s") * _NC + lax.axis_index("c")
    base = wid * _B_PER_W
    pltpu.sync_copy(idx_hbm.at[pl.ds(TC_ROWS + base, _B_PER_W)], idx_v)
    pltpu.async_copy(table_hbm.at[idx_v], rows_v, sem).wait()
    pltpu.sync_copy(rows_v, out_hbm.at[pl.ds(base, _B_PER_W)])


def _tile_lookup_kernel(idx_ref, table_ref, out_ref):
    i = pl.program_id(0)
    for j in range(B_BLK):
        idx = idx_ref[i * B_BLK + j]
        row = table_ref[pl.ds(idx, 1), :]
        out_ref[j, :, :] = jnp.broadcast_to(row, (SEQ_LEN, EMBED_DIM))


def _tile_rows_kernel(rows_ref, buf_ref, out_ref):
    del buf_ref
    out_ref[...] = jnp.broadcast_to(rows_ref[...][:, None, :], out_ref.shape)


def kernel(raag_embeddings, table):
    idx = raag_embeddings.reshape(BATCH)

    mesh = plsc.VectorSubcoreMesh(core_axis_name="c", subcore_axis_name="s")
    gathered = pl.kernel(
        _sc_gather,
        mesh=mesh,
        out_type=jax.ShapeDtypeStruct((SC_ROWS, EMBED_DIM), jnp.float32),
        scratch_types=[
            pltpu.VMEM((_B_PER_W,), jnp.int32),
            pltpu.VMEM((_B_PER_W, EMBED_DIM), jnp.float32),
            pltpu.SemaphoreType.DMA,
        ],
    )(table, idx)

    grid_spec = pltpu.PrefetchScalarGridSpec(
        num_scalar_prefetch=1,
        grid=(TC_ROWS // B_BLK,),
        in_specs=[
            pl.BlockSpec((NUM_RAAGS, EMBED_DIM), lambda i, idx_ref: (0, 0)),
        ],
        out_specs=pl.BlockSpec(
            (B_BLK, SEQ_LEN, EMBED_DIM), lambda i, idx_ref: (i, 0, 0)
        ),
    )
    buf = pl.pallas_call(
        _tile_lookup_kernel,
        grid_spec=grid_spec,
        out_shape=jax.ShapeDtypeStruct((BATCH, SEQ_LEN, EMBED_DIM), jnp.float32),
    )(idx, table)

    out = pl.pallas_call(
        _tile_rows_kernel,
        grid=(SC_ROWS // B_BLK,),
        in_specs=[
            pl.BlockSpec((B_BLK, EMBED_DIM), lambda i: (i, 0)),
            pl.BlockSpec(memory_space=pl.ANY),
        ],
        out_specs=pl.BlockSpec(
            (B_BLK, SEQ_LEN, EMBED_DIM),
            lambda i: (i + TC_ROWS // B_BLK, 0, 0),
        ),
        out_shape=jax.ShapeDtypeStruct((BATCH, SEQ_LEN, EMBED_DIM), jnp.float32),
        input_output_aliases={1: 0},
    )(gathered, buf)
    return out


# speedup vs baseline: 3.0812x; 1.2192x over previous
import jax
import jax.numpy as jnp
from jax.experimental import pallas as pl
from jax.experimental.pallas import tpu as pltpu

NUM_RAAGS = 1000
EMBED_DIM = 128
SEQ_LEN = 512
BATCH = 1024
B_BLK = 16


def _tile_kernel(idx_ref, table_ref, out_ref):
    i = pl.program_id(0)
    for j in range(B_BLK):
        idx = idx_ref[i * B_BLK + j]
        row = table_ref[pl.ds(idx, 1), :]
        out_ref[j, :, :] = jnp.broadcast_to(row, (SEQ_LEN, EMBED_DIM))


def kernel(raag_embeddings, table):
    idx = raag_embeddings.reshape(BATCH)

    grid_spec = pltpu.PrefetchScalarGridSpec(
        num_scalar_prefetch=1,
        grid=(BATCH // B_BLK,),
        in_specs=[
            pl.BlockSpec((NUM_RAAGS, EMBED_DIM), lambda i, idx_ref: (0, 0)),
        ],
        out_specs=pl.BlockSpec(
            (B_BLK, SEQ_LEN, EMBED_DIM), lambda i, idx_ref: (i, 0, 0)
        ),
    )

    out = pl.pallas_call(
        _tile_kernel,
        grid_spec=grid_spec,
        out_shape=jax.ShapeDtypeStruct((BATCH, SEQ_LEN, EMBED_DIM), jnp.float32),
    )(idx, table)
    return out
